# Initial kernel scaffold; baseline (speedup 1.0000x reference)
import jax
import jax.numpy as jnp
from jax.experimental import pallas as pl


def kernel(images):
    raise NotImplementedError("write your pallas kernel here")


# speedup vs baseline: 11.7248x; 11.7248x over previous
import functools

import jax
import jax.numpy as jnp
from jax.experimental import pallas as pl
from jax.experimental.pallas import tpu as pltpu

K = 32
BLK = 256


def _body(a_ref, b_ref, sq_ref, out_ref, dsq_ref):
    i = pl.program_id(0)

    a = a_ref[...]
    b = b_ref[...]

    af = a.astype(jnp.float32)
    row_sq = jnp.sum(af * af, axis=1, keepdims=True)
    col_sq = sq_ref[...]

    g = jax.lax.dot_general(a, b, (((1,), (0,)), ((), ())),
                            preferred_element_type=jnp.float32)
    dsq = row_sq + col_sq - 2.0 * g
    dsq = jnp.maximum(dsq, 0.0)

    n = dsq.shape[1]
    r = i * BLK + jax.lax.broadcasted_iota(jnp.int32, (BLK, n), 0)
    c = jax.lax.broadcasted_iota(jnp.int32, (BLK, n), 1)
    dsq = jnp.where(r == c, jnp.inf, dsq)
    dsq_ref[...] = dsq

    kf = jnp.float32(K)

    def sel_body(_, carry):
        acc, ans = carry
        d = dsq_ref[...]
        m = jnp.min(d, axis=1, keepdims=True)
        eq = d == m
        cnt = jnp.sum(eq.astype(jnp.float32), axis=1, keepdims=True)
        newacc = acc + cnt
        take = (acc <= kf) & (newacc > kf)
        ans = jnp.where(take, m, ans)
        dsq_ref[...] = jnp.where(eq, jnp.inf, d)
        return newacc, ans

    zeros = jnp.zeros((BLK, 1), jnp.float32)
    _, ans = jax.lax.fori_loop(0, K + 1, sel_body, (zeros, zeros))
    out_ref[...] = jnp.sqrt(ans)


@functools.partial(jax.jit, static_argnames=())
def kernel(images):
    n, d = images.shape
    a16 = images.astype(jnp.bfloat16)
    b16 = a16.T
    bf = a16.astype(jnp.float32)
    col_sq = jnp.sum(bf * bf, axis=1)[None, :]

    out = pl.pallas_call(
        _body,
        grid=(n // BLK,),
        in_specs=[
            pl.BlockSpec((BLK, d), lambda i: (i, 0)),
            pl.BlockSpec((d, n), lambda i: (0, 0)),
            pl.BlockSpec((1, n), lambda i: (0, 0)),
        ],
        out_specs=pl.BlockSpec((BLK, 1), lambda i: (i, 0)),
        out_shape=jax.ShapeDtypeStruct((n, 1), jnp.float32),
        scratch_shapes=[pltpu.VMEM((BLK, n), jnp.float32)],
    )(a16, b16, col_sq)
    return out[:, 0]


# speedup vs baseline: 35.6805x; 3.0432x over previous
import functools

import jax
import jax.numpy as jnp
from jax.experimental import pallas as pl
from jax.experimental.pallas import tpu as pltpu

K = 32
BLK = 256
MAX_FINITE_BF16_BITS = 0x7F7F


def _bits_to_bf16(bits_i32):
    return jax.lax.bitcast_convert_type(bits_i32.astype(jnp.int16), jnp.bfloat16)


def _body(a_ref, b_ref, sq_ref, out_ref, dbf_ref):
    i = pl.program_id(0)

    a = a_ref[...]
    b = b_ref[...]

    af = a.astype(jnp.float32)
    row_sq = jnp.sum(af * af, axis=1, keepdims=True)
    col_sq = sq_ref[...]

    g = jax.lax.dot_general(a, b, (((1,), (0,)), ((), ())),
                            preferred_element_type=jnp.float32)
    dsq = jnp.maximum(row_sq + col_sq - 2.0 * g, 0.0)

    n = dsq.shape[1]
    r = i * BLK + jax.lax.broadcasted_iota(jnp.int32, (BLK, n), 0)
    c = jax.lax.broadcasted_iota(jnp.int32, (BLK, n), 1)
    dbf_ref[...] = jnp.where(r == c, jnp.inf, dsq).astype(jnp.bfloat16)

    need = jnp.float32(K + 1)

    def bis(_, carry):
        lo, hi = carry
        mid = (lo + hi) >> 1
        thr = _bits_to_bf16(mid)
        x = dbf_ref[...]
        s = jnp.where(x <= thr, jnp.bfloat16(1.0), jnp.bfloat16(0.0))
        while s.shape[1] > 64:
            h = s.shape[1] // 2
            s = s[:, :h] + s[:, h:]
        cnt = jnp.sum(s.astype(jnp.float32), axis=1, keepdims=True)
        ge = cnt >= need
        return jnp.where(ge, lo, mid), jnp.where(ge, mid, hi)

    lo0 = jnp.full((BLK, 1), -1, jnp.int32)
    hi0 = jnp.full((BLK, 1), MAX_FINITE_BF16_BITS, jnp.int32)
    _, hi = jax.lax.fori_loop(0, 16, bis, (lo0, hi0))
    out_ref[...] = jnp.sqrt(_bits_to_bf16(hi).astype(jnp.float32))


@functools.partial(jax.jit, static_argnames=())
def kernel(images):
    n, d = images.shape
    a16 = images.astype(jnp.bfloat16)
    b16 = a16.T
    bf = a16.astype(jnp.float32)
    col_sq = jnp.sum(bf * bf, axis=1)[None, :]

    out = pl.pallas_call(
        _body,
        grid=(n // BLK,),
        in_specs=[
            pl.BlockSpec((BLK, d), lambda i: (i, 0)),
            pl.BlockSpec((d, n), lambda i: (0, 0)),
            pl.BlockSpec((1, n), lambda i: (0, 0)),
        ],
        out_specs=pl.BlockSpec((BLK, 1), lambda i: (i, 0)),
        out_shape=jax.ShapeDtypeStruct((n, 1), jnp.float32),
        scratch_shapes=[pltpu.VMEM((BLK, n), jnp.bfloat16)],
    )(a16, b16, col_sq)
    return out[:, 0]


# speedup vs baseline: 43.7993x; 1.2275x over previous
import functools

import jax
import jax.numpy as jnp
from jax.experimental import pallas as pl
from jax.experimental.pallas import tpu as pltpu

K = 32
BLK = 256


def _bits_to_bf16(bits_i32):
    return jax.lax.bitcast_convert_type(bits_i32.astype(jnp.int16), jnp.bfloat16)


def _bf16_to_bits(x_bf16):
    return jax.lax.bitcast_convert_type(x_bf16, jnp.int16).astype(jnp.int32)


def _fold(s, op, width):
    while s.shape[1] > width:
        h = s.shape[1] // 2
        s = op(s[:, :h], s[:, h:])
    return s


def _body(a_ref, b2_ref, sq_ref, out_ref, dbf_ref):
    i = pl.program_id(0)

    a = a_ref[...]
    b2 = b2_ref[...]

    af = a.astype(jnp.float32)
    row_sq = jnp.sum(af * af, axis=1, keepdims=True)
    col_sq = sq_ref[...]

    g2 = jax.lax.dot_general(a, b2, (((1,), (0,)), ((), ())),
                             preferred_element_type=jnp.float32)
    dsq = jnp.maximum((row_sq + col_sq) + g2, 0.0)
    dbf_ref[...] = dsq.astype(jnp.bfloat16)

    win = dbf_ref[:, pl.ds(i * BLK, BLK)]
    rl = jax.lax.broadcasted_iota(jnp.int32, (BLK, BLK), 0)
    cl = jax.lax.broadcasted_iota(jnp.int32, (BLK, BLK), 1)
    dbf_ref[:, pl.ds(i * BLK, BLK)] = jnp.where(rl == cl, jnp.inf, win)

    x = dbf_ref[...]
    rmin = _fold(x, jnp.minimum, 16)
    rmin = jnp.min(rmin.astype(jnp.float32), axis=1, keepdims=True)
    xf = jnp.where(x == jnp.inf, jnp.bfloat16(0.0), x)
    rmax = _fold(xf, jnp.maximum, 16)
    rmax = jnp.max(rmax.astype(jnp.float32), axis=1, keepdims=True)
    lo0 = _bf16_to_bits(rmin.astype(jnp.bfloat16)) - 1
    hi0 = _bf16_to_bits(rmax.astype(jnp.bfloat16))

    need = jnp.float32(K + 1)

    def cond(carry):
        lo, hi = carry
        return jnp.max(hi - lo) > 1

    def bis(carry):
        lo, hi = carry
        mid = (lo + hi) >> 1
        thr = _bits_to_bf16(mid)
        d = dbf_ref[...]
        s = jnp.where(d <= thr, jnp.bfloat16(1.0), jnp.bfloat16(0.0))
        s = _fold(s, jnp.add, 16)
        cnt = jnp.sum(s.astype(jnp.float32), axis=1, keepdims=True)
        ge = cnt >= need
        return jnp.where(ge, lo, mid), jnp.where(ge, mid, hi)

    _, hi = jax.lax.while_loop(cond, bis, (lo0, hi0))
    out_ref[...] = jnp.sqrt(_bits_to_bf16(hi).astype(jnp.float32))


@functools.partial(jax.jit, static_argnames=())
def kernel(images):
    n, d = images.shape
    a16 = images.astype(jnp.bfloat16)
    b2 = (-2.0 * a16.astype(jnp.float32)).astype(jnp.bfloat16).T
    bf = a16.astype(jnp.float32)
    col_sq = jnp.sum(bf * bf, axis=1)[None, :]

    out = pl.pallas_call(
        _body,
        grid=(n // BLK,),
        in_specs=[
            pl.BlockSpec((BLK, d), lambda i: (i, 0)),
            pl.BlockSpec((d, n), lambda i: (0, 0)),
            pl.BlockSpec((1, n), lambda i: (0, 0)),
        ],
        out_specs=pl.BlockSpec((BLK, 1), lambda i: (i, 0)),
        out_shape=jax.ShapeDtypeStruct((n, 1), jnp.float32),
        scratch_shapes=[pltpu.VMEM((BLK, n), jnp.bfloat16)],
    )(a16, b2, col_sq)
    return out[:, 0]


# speedup vs baseline: 44.2443x; 1.0102x over previous
import functools

import jax
import jax.numpy as jnp
from jax.experimental import pallas as pl
from jax.experimental.pallas import tpu as pltpu

K = 32
BLK = 256


def _bits_to_bf16(bits_i32):
    return jax.lax.bitcast_convert_type(bits_i32.astype(jnp.int16), jnp.bfloat16)


def _bf16_to_bits(x_bf16):
    return jax.lax.bitcast_convert_type(x_bf16, jnp.int16).astype(jnp.int32)


def _fold(s, op, width):
    while s.shape[1] > width:
        h = s.shape[1] // 2
        s = op(s[:, :h], s[:, h:])
    return s


def _body(a_ref, b2_ref, sq_ref, rsq_ref, out_ref, dbf_ref):
    i = pl.program_id(0)

    a = a_ref[...]
    b2 = b2_ref[...]

    row_sq = rsq_ref[...]
    col_sq = sq_ref[...]

    g2 = jax.lax.dot_general(a, b2, (((1,), (0,)), ((), ())),
                             preferred_element_type=jnp.float32)
    dsq = jnp.maximum((row_sq + col_sq) + g2, 0.0)
    dbf_ref[...] = dsq.astype(jnp.bfloat16)

    x0 = dbf_ref[...]
    rmax = _fold(x0, jnp.maximum, 16)
    rmax = jnp.max(rmax.astype(jnp.float32), axis=1, keepdims=True)

    win = dbf_ref[:, pl.ds(i * BLK, BLK)]
    rl = jax.lax.broadcasted_iota(jnp.int32, (BLK, BLK), 0)
    cl = jax.lax.broadcasted_iota(jnp.int32, (BLK, BLK), 1)
    dbf_ref[:, pl.ds(i * BLK, BLK)] = jnp.where(rl == cl, jnp.inf, win)

    x = dbf_ref[...]
    rmin = _fold(x, jnp.minimum, 16)
    rmin = jnp.min(rmin.astype(jnp.float32), axis=1, keepdims=True)
    lo0 = _bf16_to_bits(rmin.astype(jnp.bfloat16)) - 1
    hi0 = _bf16_to_bits(rmax.astype(jnp.bfloat16))

    need = jnp.float32(K + 1)

    def cond(carry):
        lo, hi = carry
        return jnp.max(hi - lo) > 1

    def bis(carry):
        lo, hi = carry
        mid = (lo + hi) >> 1
        thr = _bits_to_bf16(mid)
        d = dbf_ref[...]
        s = jnp.where(d <= thr, jnp.bfloat16(1.0), jnp.bfloat16(0.0))
        s = _fold(s, jnp.add, 16)
        cnt = jnp.sum(s.astype(jnp.float32), axis=1, keepdims=True)
        ge = cnt >= need
        return jnp.where(ge, lo, mid), jnp.where(ge, mid, hi)

    _, hi = jax.lax.while_loop(cond, bis, (lo0, hi0))
    out_ref[...] = jnp.sqrt(_bits_to_bf16(hi).astype(jnp.float32))


@functools.partial(jax.jit, static_argnames=())
def kernel(images):
    n, d = images.shape
    a16 = images.astype(jnp.bfloat16)
    b2 = (-2.0 * a16.astype(jnp.float32)).astype(jnp.bfloat16).T
    bf = a16.astype(jnp.float32)
    sq = jnp.sum(bf * bf, axis=1)
    col_sq = sq[None, :]
    row_sq = sq[:, None]

    out = pl.pallas_call(
        _body,
        grid=(n // BLK,),
        in_specs=[
            pl.BlockSpec((BLK, d), lambda i: (i, 0)),
            pl.BlockSpec((d, n), lambda i: (0, 0)),
            pl.BlockSpec((1, n), lambda i: (0, 0)),
            pl.BlockSpec((BLK, 1), lambda i: (i, 0)),
        ],
        out_specs=pl.BlockSpec((BLK, 1), lambda i: (i, 0)),
        out_shape=jax.ShapeDtypeStruct((n, 1), jnp.float32),
        scratch_shapes=[pltpu.VMEM((BLK, n), jnp.bfloat16)],
    )(a16, b2, col_sq, row_sq)
    return out[:, 0]
